# baseline (speedup 1.0000x reference)
import jax
import jax.numpy as jnp
from jax.experimental import pallas as pl
from jax.experimental.pallas import tpu as pltpu

_B, _T, _D, _H = 1024, 200, 64, 64
_S = 8


def _sig(z):
    return jnp.tanh(z * 0.5) * 0.5 + 0.5


def _lstm_body(x_ref, wx_ref, wh_ref, b_ref, out_ref, h_ref, c_ref):
    t = pl.program_id(0)

    @pl.when(t == 0)
    def _init():
        h_ref[...] = jnp.zeros_like(h_ref)
        c_ref[...] = jnp.zeros_like(c_ref)

    wx = wx_ref[...]
    wh = wh_ref[...]
    b = b_ref[0:1, :]
    dn = (((1,), (0,)), ((), ()))

    h = h_ref[...]
    c = c_ref[...]
    for s in range(_S):
        xg = jax.lax.dot_general(
            x_ref[:, s, :], wx, dn,
            preferred_element_type=jnp.float32)
        hg = jax.lax.dot_general(
            h, wh, dn, preferred_element_type=jnp.float32)
        gates = xg + hg + b
        i_g = _sig(gates[:, 0 * _H:1 * _H])
        f_g = _sig(gates[:, 1 * _H:2 * _H])
        g_g = jnp.tanh(gates[:, 2 * _H:3 * _H])
        o_g = _sig(gates[:, 3 * _H:4 * _H])
        c = f_g * c + i_g * g_g
        h = o_g * jnp.tanh(c)
        out_ref[:, s, :] = h
    h_ref[...] = h
    c_ref[...] = c


def kernel(x, W_ih, W_hh, b_ih, b_hh):
    wx = W_ih.T
    wh = W_hh.T
    b_row = jnp.broadcast_to((b_ih + b_hh)[None, :], (8, 4 * _H))

    grid = (_T // _S,)

    out = pl.pallas_call(
        _lstm_body,
        grid=grid,
        in_specs=[
            pl.BlockSpec((_B, _S, _D), lambda t: (0, t, 0)),
            pl.BlockSpec((_D, 4 * _H), lambda t: (0, 0)),
            pl.BlockSpec((_H, 4 * _H), lambda t: (0, 0)),
            pl.BlockSpec((8, 4 * _H), lambda t: (0, 0)),
        ],
        out_specs=pl.BlockSpec((_B, _S, _H), lambda t: (0, t, 0)),
        out_shape=jax.ShapeDtypeStruct((_B, _T, _H), jnp.float32),
        scratch_shapes=[
            pltpu.VMEM((_B, _H), jnp.float32),
            pltpu.VMEM((_B, _H), jnp.float32),
        ],
        compiler_params=pltpu.CompilerParams(
            dimension_semantics=("arbitrary",),
        ),
    )(x, wx, wh, b_row)

    return out


# speedup vs baseline: 2.0426x; 2.0426x over previous
import jax
import jax.numpy as jnp
from jax.experimental import pallas as pl
from jax.experimental.pallas import tpu as pltpu

_B, _T, _D, _H = 1024, 200, 64, 64
_S = 8


def _sig(z):
    return jnp.tanh(z * 0.5) * 0.5 + 0.5


def _lstm_body(x_ref, wx_ref, wh_ref, b_ref, out_ref, h_ref, c_ref):
    t = pl.program_id(0)

    @pl.when(t == 0)
    def _init():
        h_ref[...] = jnp.zeros_like(h_ref)
        c_ref[...] = jnp.zeros_like(c_ref)

    wx = wx_ref[...]
    wh = wh_ref[...]
    b = b_ref[0:1, :]
    dn = (((1,), (0,)), ((), ()))

    h = h_ref[...]
    c = c_ref[...]
    for s in range(_S):
        xg = jax.lax.dot_general(
            x_ref[:, s, :], wx, dn,
            preferred_element_type=jnp.float32)
        hg = jax.lax.dot_general(
            h, wh, dn, preferred_element_type=jnp.float32)
        gates = xg + hg + b
        i_g = _sig(gates[:, 0 * _H:1 * _H])
        f_g = _sig(gates[:, 1 * _H:2 * _H])
        g_g = jnp.tanh(gates[:, 2 * _H:3 * _H])
        o_g = _sig(gates[:, 3 * _H:4 * _H])
        c = f_g * c + i_g * g_g
        h = o_g * jnp.tanh(c)
        out_ref[:, s * _H:(s + 1) * _H] = h
    h_ref[...] = h
    c_ref[...] = c


def kernel(x, W_ih, W_hh, b_ih, b_hh):
    wx = W_ih.T
    wh = W_hh.T
    b_row = jnp.broadcast_to((b_ih + b_hh)[None, :], (8, 4 * _H))

    grid = (_T // _S,)

    out = pl.pallas_call(
        _lstm_body,
        grid=grid,
        in_specs=[
            pl.BlockSpec((_B, _S, _D), lambda t: (0, t, 0)),
            pl.BlockSpec((_D, 4 * _H), lambda t: (0, 0)),
            pl.BlockSpec((_H, 4 * _H), lambda t: (0, 0)),
            pl.BlockSpec((8, 4 * _H), lambda t: (0, 0)),
        ],
        out_specs=pl.BlockSpec((_B, _S * _H), lambda t: (0, t)),
        out_shape=jax.ShapeDtypeStruct((_B, _T * _H), jnp.float32),
        scratch_shapes=[
            pltpu.VMEM((_B, _H), jnp.float32),
            pltpu.VMEM((_B, _H), jnp.float32),
        ],
        compiler_params=pltpu.CompilerParams(
            dimension_semantics=("arbitrary",),
        ),
    )(x, wx, wh, b_row)

    return out.reshape(_B, _T, _H)
